# baseline (device time: 110074 ns/iter reference)
import jax
import jax.numpy as jnp
from jax import lax
from jax.experimental import pallas as pl
from jax.experimental.pallas import tpu as pltpu

N_DEV = 8
B_PER = 2
SQ = 256
D_MODEL = 512
H_PER = 4
DH = 64
HG = H_PER * DH
BLK = 64


def _body(x_ref, w_ref, k_ref, v_ref, out_ref, wg_ref, send_sems, recv_sems):
    my = lax.axis_index("i")
    left = lax.rem(my + N_DEV - 1, N_DEV)
    right = lax.rem(my + 1, N_DEV)

    barrier_sem = pltpu.get_barrier_semaphore()
    for nbr in (left, right):
        pl.semaphore_signal(
            barrier_sem, inc=1,
            device_id=(nbr,), device_id_type=pl.DeviceIdType.MESH,
        )
    pl.semaphore_wait(barrier_sem, 2)

    wg_ref[0] = w_ref[...]

    x2 = x_ref[...].reshape(B_PER * SQ, D_MODEL)

    qblk = lax.broadcasted_iota(jnp.int32, (SQ, SQ), 0) // BLK
    kblk = lax.broadcasted_iota(jnp.int32, (SQ, SQ), 1) // BLK
    mask = kblk <= qblk

    def compute_chunk(r):
        wq = wg_ref[r, 0:D_MODEL, :]
        wot = wg_ref[r, D_MODEL:, :]
        q2 = jnp.dot(x2, wq, preferred_element_type=jnp.float32)
        ctx_rows = []
        for b in range(B_PER):
            heads = []
            for h in range(H_PER):
                q = q2[b * SQ:(b + 1) * SQ, h * DH:(h + 1) * DH]
                k = k_ref[r, b, :, h * DH:(h + 1) * DH]
                v = v_ref[r, b, :, h * DH:(h + 1) * DH]
                s = lax.dot_general(
                    q, k, (((1,), (1,)), ((), ())),
                    preferred_element_type=jnp.float32,
                ) * 0.125
                s = jnp.where(mask, s, -1e9)
                m = jnp.max(s, axis=1, keepdims=True)
                e = jnp.exp(s - m)
                w = e / jnp.sum(e, axis=1, keepdims=True)
                heads.append(jnp.dot(w, v, preferred_element_type=jnp.float32))
            ctx_rows.append(jnp.concatenate(heads, axis=1))
        ctx2 = jnp.concatenate(ctx_rows, axis=0)
        return lax.dot_general(
            ctx2, wot, (((1,), (1,)), ((), ())),
            preferred_element_type=jnp.float32,
        )

    for h in range(N_DEV - 1):
        rdma = pltpu.make_async_remote_copy(
            src_ref=wg_ref.at[h],
            dst_ref=wg_ref.at[h + 1],
            send_sem=send_sems.at[h],
            recv_sem=recv_sems.at[h],
            device_id=(right,),
            device_id_type=pl.DeviceIdType.MESH,
        )
        rdma.start()
        contrib = compute_chunk(h).reshape(B_PER, SQ, D_MODEL)
        if h == 0:
            out_ref[...] = contrib
        else:
            out_ref[...] = out_ref[...] + contrib
        rdma.wait()

    last = compute_chunk(N_DEV - 1).reshape(B_PER, SQ, D_MODEL)
    out_ref[...] = out_ref[...] + last


def kernel(x, Wq, K_ext, V_ext, Wo):
    my = lax.axis_index("i")

    wpack = jnp.concatenate([Wq, Wo.T], axis=0)

    kb = lax.dynamic_slice_in_dim(K_ext, B_PER * my, B_PER, axis=0)
    vb = lax.dynamic_slice_in_dim(V_ext, B_PER * my, B_PER, axis=0)

    idx = jnp.mod(my - jnp.arange(N_DEV), N_DEV)
    kr = jnp.moveaxis(
        jnp.take(kb.reshape(B_PER, SQ, N_DEV, HG), idx, axis=2), 2, 0)
    vr = jnp.moveaxis(
        jnp.take(vb.reshape(B_PER, SQ, N_DEV, HG), idx, axis=2), 2, 0)

    return pl.pallas_call(
        _body,
        out_shape=jax.ShapeDtypeStruct((B_PER, SQ, D_MODEL), jnp.float32),
        in_specs=[pl.BlockSpec(memory_space=pltpu.VMEM)] * 4,
        out_specs=pl.BlockSpec(memory_space=pltpu.VMEM),
        scratch_shapes=[
            pltpu.VMEM((N_DEV, 2 * D_MODEL, HG), jnp.float32),
            pltpu.SemaphoreType.DMA((N_DEV - 1,)),
            pltpu.SemaphoreType.DMA((N_DEV - 1,)),
        ],
        compiler_params=pltpu.CompilerParams(collective_id=0),
    )(x, wpack, kr, vr)


# device time: 109737 ns/iter; 1.0031x vs baseline; 1.0031x over previous
import jax
import jax.numpy as jnp
from jax import lax
from jax.experimental import pallas as pl
from jax.experimental.pallas import tpu as pltpu

N_DEV = 8
B_PER = 2
SQ = 256
D_MODEL = 512
H_PER = 4
DH = 64
HG = H_PER * DH
BLK = 64


def _body(x_ref, wq_ref, wo_ref, k_ref, v_ref, out_ref,
          wqg_ref, wog_ref, ctx_ref,
          wq_send, wq_recv, wo_send, wo_recv):
    my = lax.axis_index("i")
    left = lax.rem(my + N_DEV - 1, N_DEV)
    right = lax.rem(my + 1, N_DEV)

    barrier_sem = pltpu.get_barrier_semaphore()
    for nbr in (left, right):
        pl.semaphore_signal(
            barrier_sem, inc=1,
            device_id=(nbr,), device_id_type=pl.DeviceIdType.MESH,
        )
    pl.semaphore_wait(barrier_sem, 2)

    wqg_ref[0] = wq_ref[...]
    wog_ref[0] = wo_ref[...]

    x2 = x_ref[...].reshape(B_PER * SQ, D_MODEL)

    qblk = lax.broadcasted_iota(jnp.int32, (SQ, SQ), 0) // BLK
    kblk = lax.broadcasted_iota(jnp.int32, (SQ, SQ), 1) // BLK
    mask = kblk <= qblk

    def compute_chunk(r):
        q2 = jnp.dot(x2, wqg_ref[r],
                     preferred_element_type=jnp.float32)
        for b in range(B_PER):
            heads = []
            for h in range(H_PER):
                q = q2[b * SQ:(b + 1) * SQ, h * DH:(h + 1) * DH]
                k = k_ref[r, b, :, h * DH:(h + 1) * DH]
                v = v_ref[r, b, :, h * DH:(h + 1) * DH]
                s = lax.dot_general(
                    q, k, (((1,), (1,)), ((), ())),
                    preferred_element_type=jnp.float32,
                )
                e = jnp.where(mask, jnp.exp(s), 0.0)
                recip = 1.0 / jnp.sum(e, axis=1, keepdims=True)
                ctx = jnp.dot(e, v,
                              preferred_element_type=jnp.float32) * recip
                heads.append(ctx)
            ctx_b = jnp.concatenate(heads, axis=1)
            ctx_ref[b * SQ:(b + 1) * SQ, r * HG:(r + 1) * HG] = ctx_b

    for h in range(N_DEV - 1):
        rdma_wq = pltpu.make_async_remote_copy(
            src_ref=wqg_ref.at[h], dst_ref=wqg_ref.at[h + 1],
            send_sem=wq_send.at[h], recv_sem=wq_recv.at[h],
            device_id=(right,), device_id_type=pl.DeviceIdType.MESH,
        )
        rdma_wo = pltpu.make_async_remote_copy(
            src_ref=wog_ref.at[h], dst_ref=wog_ref.at[h + 1],
            send_sem=wo_send.at[h], recv_sem=wo_recv.at[h],
            device_id=(right,), device_id_type=pl.DeviceIdType.MESH,
        )
        rdma_wq.start()
        rdma_wo.start()
        compute_chunk(h)
        rdma_wq.wait()
        rdma_wo.wait()
    compute_chunk(N_DEV - 1)

    wo_all = wog_ref[...].reshape(N_DEV * HG, D_MODEL)
    out2 = jnp.dot(ctx_ref[...], wo_all,
                   preferred_element_type=jnp.float32)
    out_ref[...] = out2.reshape(B_PER, SQ, D_MODEL)


def kernel(x, Wq, K_ext, V_ext, Wo):
    my = lax.axis_index("i")

    wq_scaled = Wq * 0.125

    kb = lax.dynamic_slice_in_dim(K_ext, B_PER * my, B_PER, axis=0)
    vb = lax.dynamic_slice_in_dim(V_ext, B_PER * my, B_PER, axis=0)

    idx = jnp.mod(my - jnp.arange(N_DEV), N_DEV)
    kr = jnp.moveaxis(
        jnp.take(kb.reshape(B_PER, SQ, N_DEV, HG), idx, axis=2), 2, 0)
    vr = jnp.moveaxis(
        jnp.take(vb.reshape(B_PER, SQ, N_DEV, HG), idx, axis=2), 2, 0)

    return pl.pallas_call(
        _body,
        out_shape=jax.ShapeDtypeStruct((B_PER, SQ, D_MODEL), jnp.float32),
        in_specs=[pl.BlockSpec(memory_space=pltpu.VMEM)] * 5,
        out_specs=pl.BlockSpec(memory_space=pltpu.VMEM),
        scratch_shapes=[
            pltpu.VMEM((N_DEV, D_MODEL, HG), jnp.float32),
            pltpu.VMEM((N_DEV, HG, D_MODEL), jnp.float32),
            pltpu.VMEM((B_PER * SQ, N_DEV * HG), jnp.float32),
            pltpu.SemaphoreType.DMA((N_DEV - 1,)),
            pltpu.SemaphoreType.DMA((N_DEV - 1,)),
            pltpu.SemaphoreType.DMA((N_DEV - 1,)),
            pltpu.SemaphoreType.DMA((N_DEV - 1,)),
        ],
        compiler_params=pltpu.CompilerParams(collective_id=0),
    )(x, wq_scaled, Wo, kr, vr)


# device time: 47703 ns/iter; 2.3075x vs baseline; 2.3004x over previous
import jax
import jax.numpy as jnp
from jax import lax
from jax.experimental import pallas as pl
from jax.experimental.pallas import tpu as pltpu

N_DEV = 8
B_PER = 2
SQ = 256
D_MODEL = 512
H_PER = 4
DH = 64
HG = H_PER * DH
BLK = 64

_R_SRC, _R_DST = (0, 1, 2), (1, 2, 3)
_L_SRC, _L_DST = (0, 7, 6), (7, 6, 5)


def _body(x_ref, w_ref, k_ref, v_ref, out_ref, wg_ref,
          r_send, r_recv, l_send, l_recv, z_send, z_recv):
    my = lax.axis_index("i")
    left = lax.rem(my + N_DEV - 1, N_DEV)
    right = lax.rem(my + 1, N_DEV)
    zpeer = lax.rem(my + 4, N_DEV)

    barrier_sem = pltpu.get_barrier_semaphore()
    for nbr in (left, right, zpeer):
        pl.semaphore_signal(
            barrier_sem, inc=1,
            device_id=(nbr,), device_id_type=pl.DeviceIdType.MESH,
        )
    pl.semaphore_wait(barrier_sem, 3)

    wg_ref[0] = w_ref[...]

    x2b = x_ref[...].reshape(B_PER * SQ, D_MODEL).astype(jnp.bfloat16)

    qblk = lax.broadcasted_iota(jnp.int32, (SQ, SQ), 0) // BLK
    kblk = lax.broadcasted_iota(jnp.int32, (SQ, SQ), 1) // BLK
    mask = kblk <= qblk

    def compute_chunk(r, first=False):
        wq = wg_ref[r, 0:D_MODEL, :]
        wot = wg_ref[r, D_MODEL:, :]
        q2 = jnp.dot(x2b, wq, preferred_element_type=jnp.float32)
        ctx_rows = []
        for b in range(B_PER):
            heads = []
            for h in range(H_PER):
                q = q2[b * SQ:(b + 1) * SQ, h * DH:(h + 1) * DH]
                k = k_ref[r, b, :, h * DH:(h + 1) * DH]
                v = v_ref[r, b, :, h * DH:(h + 1) * DH]
                s = lax.dot_general(
                    q, k, (((1,), (1,)), ((), ())),
                    preferred_element_type=jnp.float32,
                )
                e = jnp.where(mask, jnp.exp(s), 0.0)
                recip = 1.0 / jnp.sum(e, axis=1, keepdims=True)
                ctx = jnp.dot(e, v,
                              preferred_element_type=jnp.float32) * recip
                heads.append(ctx)
            ctx_rows.append(jnp.concatenate(heads, axis=1))
        ctx2 = jnp.concatenate(ctx_rows, axis=0).astype(jnp.bfloat16)
        contrib = lax.dot_general(
            ctx2, wot, (((1,), (1,)), ((), ())),
            preferred_element_type=jnp.float32,
        ).reshape(B_PER, SQ, D_MODEL)
        if first:
            out_ref[...] = contrib
        else:
            out_ref[...] = out_ref[...] + contrib

    def stream_copy(src_slot, dst_slot, send_sem, recv_sem, peer):
        return pltpu.make_async_remote_copy(
            src_ref=wg_ref.at[src_slot], dst_ref=wg_ref.at[dst_slot],
            send_sem=send_sem, recv_sem=recv_sem,
            device_id=(peer,), device_id_type=pl.DeviceIdType.MESH,
        )

    for t in range(3):
        rr = stream_copy(_R_SRC[t], _R_DST[t], r_send.at[t], r_recv.at[t],
                         right)
        rl = stream_copy(_L_SRC[t], _L_DST[t], l_send.at[t], l_recv.at[t],
                         left)
        rr.start()
        rl.start()
        rz = None
        if t == 0:
            rz = stream_copy(0, 4, z_send.at[0], z_recv.at[0], zpeer)
            rz.start()
        if t == 0:
            compute_chunk(0, first=True)
        elif t == 1:
            compute_chunk(1)
            compute_chunk(7)
            compute_chunk(4)
        else:
            compute_chunk(2)
            compute_chunk(6)
        rr.wait()
        rl.wait()
        if rz is not None:
            rz.wait()

    compute_chunk(3)
    compute_chunk(5)


def kernel(x, Wq, K_ext, V_ext, Wo):
    my = lax.axis_index("i")

    wpack = jnp.concatenate([Wq * 0.125, Wo.T], axis=0).astype(jnp.bfloat16)

    kb = lax.dynamic_slice_in_dim(K_ext, B_PER * my, B_PER, axis=0)
    vb = lax.dynamic_slice_in_dim(V_ext, B_PER * my, B_PER, axis=0)

    idx = jnp.mod(my - jnp.arange(N_DEV), N_DEV)
    kr = jnp.moveaxis(
        jnp.take(kb.reshape(B_PER, SQ, N_DEV, HG), idx, axis=2), 2, 0)
    vr = jnp.moveaxis(
        jnp.take(vb.reshape(B_PER, SQ, N_DEV, HG), idx, axis=2), 2, 0)

    return pl.pallas_call(
        _body,
        out_shape=jax.ShapeDtypeStruct((B_PER, SQ, D_MODEL), jnp.float32),
        in_specs=[pl.BlockSpec(memory_space=pltpu.VMEM)] * 4,
        out_specs=pl.BlockSpec(memory_space=pltpu.VMEM),
        scratch_shapes=[
            pltpu.VMEM((N_DEV, 2 * D_MODEL, HG), jnp.bfloat16),
            pltpu.SemaphoreType.DMA((3,)),
            pltpu.SemaphoreType.DMA((3,)),
            pltpu.SemaphoreType.DMA((3,)),
            pltpu.SemaphoreType.DMA((3,)),
            pltpu.SemaphoreType.DMA((1,)),
            pltpu.SemaphoreType.DMA((1,)),
        ],
        compiler_params=pltpu.CompilerParams(collective_id=0),
    )(x, wpack, kr, vr)
